# Initial kernel scaffold; baseline (speedup 1.0000x reference)
import jax
import jax.numpy as jnp
from jax.experimental import pallas as pl


def kernel(s, v, edge_index, rbf, unit, W_f1, b_f1, W_f2, b_f2, W_s1, b_s1, W_s2, b_s2, U_w, V_w, W_u1, b_u1, W_u2, b_u2):
    raise NotImplementedError("write your pallas kernel here")


# speedup vs baseline: 1.1074x; 1.1074x over previous
import functools

import jax
import jax.numpy as jnp
from jax.experimental import pallas as pl
from jax.experimental.pallas import tpu as pltpu


def _silu(x):
    return x * jax.nn.sigmoid(x)


def _filter_body(rbf_ref, w1_ref, b1_ref, w2_ref, b2_ref, f_ref):
    h = _silu(jnp.dot(rbf_ref[...], w1_ref[...],
                      preferred_element_type=jnp.float32) + b1_ref[...])
    f_ref[...] = jnp.dot(h, w2_ref[...],
                         preferred_element_type=jnp.float32) + b2_ref[...]


def _filter_mlp(rbf, W_f1, b_f1, W_f2, b_f2, bE=2000):
    E = rbf.shape[0]
    grid = (E // bE,)
    return pl.pallas_call(
        _filter_body,
        grid=grid,
        in_specs=[
            pl.BlockSpec((bE, rbf.shape[1]), lambda i: (i, 0)),
            pl.BlockSpec(W_f1.shape, lambda i: (0, 0)),
            pl.BlockSpec(b_f1.shape, lambda i: (0,)),
            pl.BlockSpec(W_f2.shape, lambda i: (0, 0)),
            pl.BlockSpec(b_f2.shape, lambda i: (0,)),
        ],
        out_specs=pl.BlockSpec((bE, W_f2.shape[1]), lambda i: (i, 0)),
        out_shape=jax.ShapeDtypeStruct((E, W_f2.shape[1]), jnp.float32),
    )(rbf, W_f1, b_f1, W_f2, b_f2)


def _tables_body(s_ref, v_ref, w1_ref, b1_ref, w2_ref, b2_ref, t_ref, g_ref):
    h = _silu(jnp.dot(s_ref[...], w1_ref[...],
                      preferred_element_type=jnp.float32) + b1_ref[...])
    t = jnp.dot(h, w2_ref[...], preferred_element_type=jnp.float32) + b2_ref[...]
    t_ref[...] = t
    H = s_ref.shape[1]
    t_vv = t[:, H:2 * H]
    g_ref[...] = t_vv[:, None, :] * v_ref[...]


def _node_tables(s, v, W_s1, b_s1, W_s2, b_s2, bN=400):
    N, H = s.shape
    grid = (N // bN,)
    return pl.pallas_call(
        _tables_body,
        grid=grid,
        in_specs=[
            pl.BlockSpec((bN, H), lambda i: (i, 0)),
            pl.BlockSpec((bN, 3, H), lambda i: (i, 0, 0)),
            pl.BlockSpec(W_s1.shape, lambda i: (0, 0)),
            pl.BlockSpec(b_s1.shape, lambda i: (0,)),
            pl.BlockSpec(W_s2.shape, lambda i: (0, 0)),
            pl.BlockSpec(b_s2.shape, lambda i: (0,)),
        ],
        out_specs=[
            pl.BlockSpec((bN, 3 * H), lambda i: (i, 0)),
            pl.BlockSpec((bN, 3, H), lambda i: (i, 0, 0)),
        ],
        out_shape=[
            jax.ShapeDtypeStruct((N, 3 * H), jnp.float32),
            jax.ShapeDtypeStruct((N, 3, H), jnp.float32),
        ],
    )(s, v, W_s1, b_s1, W_s2, b_s2)


def _update_body(s_ref, v_ref, ds_ref, dv_ref, uw_ref, vw_ref,
                 wu1_ref, bu1_ref, wu2_ref, bu2_ref, s_out_ref, v_out_ref):
    bN, _, H = v_ref.shape
    s1 = s_ref[...] + ds_ref[...]
    v1 = v_ref[...] + dv_ref[...]
    v1f = v1.reshape(bN * 3, H)
    v_u = jnp.dot(v1f, uw_ref[...], preferred_element_type=jnp.float32)
    v_v = jnp.dot(v1f, vw_ref[...], preferred_element_type=jnp.float32)
    v_u = v_u.reshape(bN, 3, H)
    v_v = v_v.reshape(bN, 3, H)
    sq = jnp.sum(v_v * v_v, axis=1)
    v_norm = jnp.sqrt(sq)
    upd_in = jnp.concatenate([s1, v_norm], axis=-1)
    h = _silu(jnp.dot(upd_in, wu1_ref[...],
                      preferred_element_type=jnp.float32) + bu1_ref[...])
    out = jnp.dot(h, wu2_ref[...], preferred_element_type=jnp.float32) + bu2_ref[...]
    a = out[:, :H]
    b = out[:, H:2 * H]
    c = out[:, 2 * H:]
    inner = jnp.sum(v_u * v_v, axis=1)
    s_out_ref[...] = s1 + a + b * inner
    v_out_ref[...] = v1 + c[:, None, :] * v_u


def _update_phase(s, v, ds, dv, U_w, V_w, W_u1, b_u1, W_u2, b_u2, bN=400):
    N, H = s.shape
    grid = (N // bN,)
    return pl.pallas_call(
        _update_body,
        grid=grid,
        in_specs=[
            pl.BlockSpec((bN, H), lambda i: (i, 0)),
            pl.BlockSpec((bN, 3, H), lambda i: (i, 0, 0)),
            pl.BlockSpec((bN, H), lambda i: (i, 0)),
            pl.BlockSpec((bN, 3, H), lambda i: (i, 0, 0)),
            pl.BlockSpec(U_w.shape, lambda i: (0, 0)),
            pl.BlockSpec(V_w.shape, lambda i: (0, 0)),
            pl.BlockSpec(W_u1.shape, lambda i: (0, 0)),
            pl.BlockSpec(b_u1.shape, lambda i: (0,)),
            pl.BlockSpec(W_u2.shape, lambda i: (0, 0)),
            pl.BlockSpec(b_u2.shape, lambda i: (0,)),
        ],
        out_specs=[
            pl.BlockSpec((bN, H), lambda i: (i, 0)),
            pl.BlockSpec((bN, 3, H), lambda i: (i, 0, 0)),
        ],
        out_shape=[
            jax.ShapeDtypeStruct((N, H), jnp.float32),
            jax.ShapeDtypeStruct((N, 3, H), jnp.float32),
        ],
    )(s, v, ds, dv, U_w, V_w, W_u1, b_u1, W_u2, b_u2)


def kernel(s, v, edge_index, rbf, unit,
           W_f1, b_f1, W_f2, b_f2,
           W_s1, b_s1, W_s2, b_s2,
           U_w, V_w, W_u1, b_u1, W_u2, b_u2):
    N, H = s.shape
    src = edge_index[0]
    dst = edge_index[1]

    f = _filter_mlp(rbf, W_f1, b_f1, W_f2, b_f2)
    t, g = _node_tables(s, v, W_s1, b_s1, W_s2, b_s2)

    f_ds, f_vv, f_vr = f[:, :H], f[:, H:2 * H], f[:, 2 * H:]
    t_src = t[src]
    ds_e = f_ds * t_src[:, :H]
    m_vr = f_vr * t_src[:, 2 * H:]
    g_src = g[src]
    dv_e = f_vv[:, None, :] * g_src + m_vr[:, None, :] * unit[:, :, None]
    ds = jax.ops.segment_sum(ds_e, dst, num_segments=N)
    dv = jax.ops.segment_sum(dv_e, dst, num_segments=N)

    return _update_phase(s, v, ds, dv, U_w, V_w, W_u1, b_u1, W_u2, b_u2)


# speedup vs baseline: 8.3984x; 7.5838x over previous
import functools

import jax
import jax.numpy as jnp
from jax import lax
from jax.experimental import pallas as pl
from jax.experimental.pallas import tpu as pltpu
from jax.experimental.pallas import tpu_sc as plsc


def _silu(x):
    return x * jax.nn.sigmoid(x)


def _filter_body(rbf_ref, w1_ref, b1_ref, w2_ref, b2_ref, fa_ref, fb_ref):
    h = _silu(jnp.dot(rbf_ref[...], w1_ref[...],
                      preferred_element_type=jnp.float32) + b1_ref[...])
    f = jnp.dot(h, w2_ref[...], preferred_element_type=jnp.float32) + b2_ref[...]
    fa_ref[0] = f[:, 0:64]
    fa_ref[1] = f[:, 64:128]
    fb_ref[0] = f[:, 128:256]
    fb_ref[1] = f[:, 256:384]


def _filter_mlp(rbf, W_f1, b_f1, W_f2p, b_f2p, bE=2048):
    E, R = rbf.shape
    grid = (E // bE,)
    return pl.pallas_call(
        _filter_body,
        grid=grid,
        in_specs=[
            pl.BlockSpec((bE, R), lambda i: (i, 0)),
            pl.BlockSpec(W_f1.shape, lambda i: (0, 0)),
            pl.BlockSpec(b_f1.shape, lambda i: (0,)),
            pl.BlockSpec(W_f2p.shape, lambda i: (0, 0)),
            pl.BlockSpec(b_f2p.shape, lambda i: (0,)),
        ],
        out_specs=[
            pl.BlockSpec((2, bE, 64), lambda i: (0, i, 0)),
            pl.BlockSpec((2, bE, 128), lambda i: (0, i, 0)),
        ],
        out_shape=[
            jax.ShapeDtypeStruct((2, E, 64), jnp.float32),
            jax.ShapeDtypeStruct((2, E, 128), jnp.float32),
        ],
    )(rbf, W_f1, b_f1, W_f2p, b_f2p)


def _tables_body(s_ref, v_ref, w1_ref, b1_ref, w2_ref, b2_ref, t1_ref, t2_ref):
    bN = s_ref.shape[0]
    h = _silu(jnp.dot(s_ref[...], w1_ref[...],
                      preferred_element_type=jnp.float32) + b1_ref[...])
    t = jnp.dot(h, w2_ref[...], preferred_element_type=jnp.float32) + b2_ref[...]
    v = v_ref[...]
    pad = jnp.zeros((bN, 64), jnp.float32)
    for c in range(2):
        hs = pl.ds(64 * c, 64)
        tds = t[:, 64 * c:64 * c + 64]
        tvv = t[:, 128 + 64 * c:128 + 64 * c + 64]
        tvr = t[:, 256 + 64 * c:256 + 64 * c + 64]
        g0 = tvv * v[:, 0, 64 * c:64 * c + 64]
        g1 = tvv * v[:, 1, 64 * c:64 * c + 64]
        g2 = tvv * v[:, 2, 64 * c:64 * c + 64]
        t1_ref[c] = jnp.concatenate([tds, g0, tvr, pad], axis=-1)
        t2_ref[c] = jnp.concatenate([g1, g2, tvr, pad], axis=-1)


def _node_tables(s, v, W_s1, b_s1, W_s2, b_s2, bN=400):
    N, H = s.shape
    grid = (N // bN,)
    return pl.pallas_call(
        _tables_body,
        grid=grid,
        in_specs=[
            pl.BlockSpec((bN, H), lambda i: (i, 0)),
            pl.BlockSpec((bN, 3, H), lambda i: (i, 0, 0)),
            pl.BlockSpec(W_s1.shape, lambda i: (0, 0)),
            pl.BlockSpec(b_s1.shape, lambda i: (0,)),
            pl.BlockSpec(W_s2.shape, lambda i: (0, 0)),
            pl.BlockSpec(b_s2.shape, lambda i: (0,)),
        ],
        out_specs=[
            pl.BlockSpec((2, bN, 256), lambda i: (0, i, 0)),
            pl.BlockSpec((2, bN, 256), lambda i: (0, i, 0)),
        ],
        out_shape=[
            jax.ShapeDtypeStruct((2, N, 256), jnp.float32),
            jax.ShapeDtypeStruct((2, N, 256), jnp.float32),
        ],
    )(s, v, W_s1, b_s1, W_s2, b_s2)


_K = 64


def _sc_body(N, Np, E, fa_hbm, fb_hbm, t1_hbm, t2_hbm, src_hbm, dst_hbm,
             unitT_hbm, zeros_hbm, out1_hbm, out2_hbm,
             acc, sbuf, dbuf, gidx, ubuf, fabuf, fbbuf, gbuf, obuf, sem):
    c = lax.axis_index("c")
    sid = lax.axis_index("s")
    K = _K
    ept = E // 16
    nwin = ept // K
    tile_lo = sid * ept
    coff_e = c * E

    rows = Np // 16
    row_lo = sid * rows
    cNp = c * Np

    cN_vec = jnp.full((16,), c * N, jnp.int32)

    def zero_acc():
        pltpu.sync_copy(zeros_hbm.at[pl.ds(row_lo, rows)],
                        acc.at[pl.ds(row_lo, rows)])
        plsc.subcore_barrier()

    def dump_acc(out_hbm):
        plsc.subcore_barrier()
        pltpu.sync_copy(acc.at[pl.ds(row_lo, rows)],
                        out_hbm.at[pl.ds(cNp + row_lo, rows)])
        plsc.subcore_barrier()

    zero_acc()

    def ebody1(k, carry):
        u0 = plsc.load_gather(ubuf, [jnp.full((16,), 0, jnp.int32) + k])
        for j in range(4):
            sl = pl.ds(j * 16, 16)
            sh = pl.ds(64 + j * 16, 16)
            tds = gbuf[k, sl]
            g0 = gbuf[k, sh]
            tvr = gbuf[k, pl.ds(128 + j * 16, 16)]
            fds = fabuf[k, sl]
            fvv = fbbuf[k, sl]
            fvr = fbbuf[k, sh]
            mvr = fvr * tvr
            obuf[k, sl] = fds * tds
            obuf[k, sh] = fvv * g0 + mvr * u0
        return carry

    def wbody1(w, carry):
        base = tile_lo + w * K
        pltpu.sync_copy(src_hbm.at[pl.ds(base, K)], sbuf)
        pltpu.sync_copy(dst_hbm.at[pl.ds(base, K)], dbuf)
        pltpu.sync_copy(fa_hbm.at[pl.ds(coff_e + base, K)], fabuf)
        pltpu.sync_copy(fb_hbm.at[pl.ds(coff_e + base, K)], fbbuf)
        pltpu.sync_copy(unitT_hbm.at[pl.ds(base, K)], ubuf.at[pl.ds(0, K)])
        for i in range(K // 16):
            sl = pl.ds(i * 16, 16)
            gidx[sl] = sbuf[sl] + cN_vec
        pltpu.async_copy(t1_hbm.at[gidx], gbuf, sem).wait()
        lax.fori_loop(0, K, ebody1, 0)
        pltpu.sync_copy(obuf, acc.at[dbuf], add=True)
        return carry

    lax.fori_loop(0, nwin, wbody1, 0)
    dump_acc(out1_hbm)

    zero_acc()

    def ebody2(k, carry):
        u1 = plsc.load_gather(ubuf, [jnp.full((16,), 0, jnp.int32) + k])
        u2 = plsc.load_gather(ubuf, [jnp.full((16,), K, jnp.int32) + k])
        for j in range(4):
            sl = pl.ds(j * 16, 16)
            sh = pl.ds(64 + j * 16, 16)
            g1 = gbuf[k, sl]
            g2 = gbuf[k, sh]
            tvr = gbuf[k, pl.ds(128 + j * 16, 16)]
            fvv = fbbuf[k, sl]
            fvr = fbbuf[k, sh]
            mvr = fvr * tvr
            obuf[k, sl] = fvv * g1 + mvr * u1
            obuf[k, sh] = fvv * g2 + mvr * u2
        return carry

    def wbody2(w, carry):
        base = tile_lo + w * K
        pltpu.sync_copy(src_hbm.at[pl.ds(base, K)], sbuf)
        pltpu.sync_copy(dst_hbm.at[pl.ds(base, K)], dbuf)
        pltpu.sync_copy(fb_hbm.at[pl.ds(coff_e + base, K)], fbbuf)
        pltpu.sync_copy(unitT_hbm.at[pl.ds(E + base, K)], ubuf.at[pl.ds(0, K)])
        pltpu.sync_copy(unitT_hbm.at[pl.ds(2 * E + base, K)], ubuf.at[pl.ds(K, K)])
        for i in range(K // 16):
            sl = pl.ds(i * 16, 16)
            gidx[sl] = sbuf[sl] + cN_vec
        pltpu.async_copy(t2_hbm.at[gidx], gbuf, sem).wait()
        lax.fori_loop(0, K, ebody2, 0)
        pltpu.sync_copy(obuf, acc.at[dbuf], add=True)
        return carry

    lax.fori_loop(0, nwin, wbody2, 0)
    dump_acc(out2_hbm)


def _sc_scatter(fa, fb, t1, t2, src, dst, unitT, zeros, N, Np, E):
    mesh = plsc.VectorSubcoreMesh(core_axis_name="c", subcore_axis_name="s")
    K = _K
    kfn = functools.partial(
        pl.kernel,
        out_type=[
            jax.ShapeDtypeStruct((2 * Np, 128), jnp.float32),
            jax.ShapeDtypeStruct((2 * Np, 128), jnp.float32),
        ],
        mesh=mesh,
        scratch_types=[
            pltpu.VMEM_SHARED((Np, 128), jnp.float32),
            pltpu.VMEM((K,), jnp.int32),
            pltpu.VMEM((K,), jnp.int32),
            pltpu.VMEM((K,), jnp.int32),
            pltpu.VMEM((2 * K,), jnp.float32),
            pltpu.VMEM((K, 64), jnp.float32),
            pltpu.VMEM((K, 128), jnp.float32),
            pltpu.VMEM((K, 256), jnp.float32),
            pltpu.VMEM((K, 128), jnp.float32),
            pltpu.SemaphoreType.DMA,
        ],
        compiler_params=pltpu.CompilerParams(needs_layout_passes=False),
    )(functools.partial(_sc_body, N, Np, E))
    return kfn(fa, fb, t1, t2, src, dst, unitT, zeros)


def _update_body(s_ref, v_ref, o1_ref, o2_ref, uw_ref, vw_ref,
                 wu1_ref, bu1_ref, wu2_ref, bu2_ref, s_out_ref, v_out_ref):
    bN, _, H = v_ref.shape
    ds = jnp.concatenate([o1_ref[0][:, 0:64], o1_ref[1][:, 0:64]], axis=-1)
    dv0 = jnp.concatenate([o1_ref[0][:, 64:128], o1_ref[1][:, 64:128]], axis=-1)
    dv1 = jnp.concatenate([o2_ref[0][:, 0:64], o2_ref[1][:, 0:64]], axis=-1)
    dv2 = jnp.concatenate([o2_ref[0][:, 64:128], o2_ref[1][:, 64:128]], axis=-1)
    dv = jnp.concatenate([dv0[:, None, :], dv1[:, None, :], dv2[:, None, :]],
                         axis=1)
    s1 = s_ref[...] + ds
    v1 = v_ref[...] + dv
    v1f = v1.reshape(bN * 3, H)
    v_u = jnp.dot(v1f, uw_ref[...], preferred_element_type=jnp.float32)
    v_v = jnp.dot(v1f, vw_ref[...], preferred_element_type=jnp.float32)
    v_u = v_u.reshape(bN, 3, H)
    v_v = v_v.reshape(bN, 3, H)
    v_norm = jnp.sqrt(jnp.sum(v_v * v_v, axis=1))
    upd_in = jnp.concatenate([s1, v_norm], axis=-1)
    h = _silu(jnp.dot(upd_in, wu1_ref[...],
                      preferred_element_type=jnp.float32) + bu1_ref[...])
    out = jnp.dot(h, wu2_ref[...], preferred_element_type=jnp.float32) + bu2_ref[...]
    a = out[:, :H]
    b = out[:, H:2 * H]
    cc = out[:, 2 * H:]
    inner = jnp.sum(v_u * v_v, axis=1)
    s_out_ref[...] = s1 + a + b * inner
    v_out_ref[...] = v1 + cc[:, None, :] * v_u


def _update_phase(s, v, o1, o2, U_w, V_w, W_u1, b_u1, W_u2, b_u2, bN=400):
    N, H = s.shape
    grid = (N // bN,)
    return pl.pallas_call(
        _update_body,
        grid=grid,
        in_specs=[
            pl.BlockSpec((bN, H), lambda i: (i, 0)),
            pl.BlockSpec((bN, 3, H), lambda i: (i, 0, 0)),
            pl.BlockSpec((2, bN, 128), lambda i: (0, i, 0)),
            pl.BlockSpec((2, bN, 128), lambda i: (0, i, 0)),
            pl.BlockSpec(U_w.shape, lambda i: (0, 0)),
            pl.BlockSpec(V_w.shape, lambda i: (0, 0)),
            pl.BlockSpec(W_u1.shape, lambda i: (0, 0)),
            pl.BlockSpec(b_u1.shape, lambda i: (0,)),
            pl.BlockSpec(W_u2.shape, lambda i: (0, 0)),
            pl.BlockSpec(b_u2.shape, lambda i: (0,)),
        ],
        out_specs=[
            pl.BlockSpec((bN, H), lambda i: (i, 0)),
            pl.BlockSpec((bN, 3, H), lambda i: (i, 0, 0)),
        ],
        out_shape=[
            jax.ShapeDtypeStruct((N, H), jnp.float32),
            jax.ShapeDtypeStruct((N, 3, H), jnp.float32),
        ],
    )(s, v, o1, o2, U_w, V_w, W_u1, b_u1, W_u2, b_u2)


def kernel(s, v, edge_index, rbf, unit,
           W_f1, b_f1, W_f2, b_f2,
           W_s1, b_s1, W_s2, b_s2,
           U_w, V_w, W_u1, b_u1, W_u2, b_u2):
    N, H = s.shape
    E = edge_index.shape[1]
    src = edge_index[0]
    dst = edge_index[1]

    perm = jnp.concatenate([
        jnp.arange(0, 128), jnp.arange(128, 192), jnp.arange(256, 320),
        jnp.arange(192, 256), jnp.arange(320, 384)])
    W_f2p = W_f2[:, perm]
    b_f2p = b_f2[perm]

    Np = 10240
    Ep = 327680
    npad = Ep - E
    rbf_p = jnp.pad(rbf, ((0, npad), (0, 0)))
    src_p = jnp.concatenate([src, jnp.arange(npad, dtype=jnp.int32) % N])
    dst_p = jnp.concatenate(
        [dst, N + (jnp.arange(npad, dtype=jnp.int32) % (Np - N))])
    unit_p = jnp.pad(unit, ((0, npad), (0, 0)))

    fa, fb = _filter_mlp(rbf_p, W_f1, b_f1, W_f2p, b_f2p)
    t1, t2 = _node_tables(s, v, W_s1, b_s1, W_s2, b_s2)

    fa = fa.reshape(2 * Ep, 64)
    fb = fb.reshape(2 * Ep, 128)
    t1 = t1.reshape(2 * N, 256)
    t2 = t2.reshape(2 * N, 256)
    unitT = unit_p.T.reshape(3 * Ep)
    zeros = jnp.zeros((Np, 128), jnp.float32)

    o1, o2 = _sc_scatter(fa, fb, t1, t2, src_p, dst_p, unitT, zeros, N, Np, Ep)
    o1 = o1.reshape(2, Np, 128)
    o2 = o2.reshape(2, Np, 128)

    return _update_phase(s, v, o1, o2, U_w, V_w, W_u1, b_u1, W_u2, b_u2)


# speedup vs baseline: 12.1495x; 1.4466x over previous
import functools

import jax
import jax.numpy as jnp
from jax import lax
from jax.experimental import pallas as pl
from jax.experimental.pallas import tpu as pltpu
from jax.experimental.pallas import tpu_sc as plsc


def _silu(x):
    return x * jax.nn.sigmoid(x)


def _filter_body(rbf_ref, w1_ref, b1_ref, w2_ref, b2_ref, fa_ref, fb_ref):
    h = _silu(jnp.dot(rbf_ref[...], w1_ref[...],
                      preferred_element_type=jnp.float32) + b1_ref[...])
    f = jnp.dot(h, w2_ref[...], preferred_element_type=jnp.float32) + b2_ref[...]
    fa_ref[0] = f[:, 0:64]
    fa_ref[1] = f[:, 64:128]
    fb_ref[0] = f[:, 128:256]
    fb_ref[1] = f[:, 256:384]


def _filter_mlp(rbf, W_f1, b_f1, W_f2p, b_f2p, bE=2048):
    E, R = rbf.shape
    grid = (E // bE,)
    return pl.pallas_call(
        _filter_body,
        grid=grid,
        in_specs=[
            pl.BlockSpec((bE, R), lambda i: (i, 0)),
            pl.BlockSpec(W_f1.shape, lambda i: (0, 0)),
            pl.BlockSpec(b_f1.shape, lambda i: (0,)),
            pl.BlockSpec(W_f2p.shape, lambda i: (0, 0)),
            pl.BlockSpec(b_f2p.shape, lambda i: (0,)),
        ],
        out_specs=[
            pl.BlockSpec((2, bE, 64), lambda i: (0, i, 0)),
            pl.BlockSpec((2, bE, 128), lambda i: (0, i, 0)),
        ],
        out_shape=[
            jax.ShapeDtypeStruct((2, E, 64), jnp.float32),
            jax.ShapeDtypeStruct((2, E, 128), jnp.float32),
        ],
    )(rbf, W_f1, b_f1, W_f2p, b_f2p)


def _tables_body(s_ref, v_ref, w1_ref, b1_ref, w2_ref, b2_ref, t1_ref, t2_ref):
    bN = s_ref.shape[0]
    h = _silu(jnp.dot(s_ref[...], w1_ref[...],
                      preferred_element_type=jnp.float32) + b1_ref[...])
    t = jnp.dot(h, w2_ref[...], preferred_element_type=jnp.float32) + b2_ref[...]
    v = v_ref[...]
    pad = jnp.zeros((bN, 64), jnp.float32)
    for c in range(2):
        hs = pl.ds(64 * c, 64)
        tds = t[:, 64 * c:64 * c + 64]
        tvv = t[:, 128 + 64 * c:128 + 64 * c + 64]
        tvr = t[:, 256 + 64 * c:256 + 64 * c + 64]
        g0 = tvv * v[:, 0, 64 * c:64 * c + 64]
        g1 = tvv * v[:, 1, 64 * c:64 * c + 64]
        g2 = tvv * v[:, 2, 64 * c:64 * c + 64]
        t1_ref[c] = jnp.concatenate([tds, g0, tvr, pad], axis=-1)
        t2_ref[c] = jnp.concatenate([g1, g2, tvr, pad], axis=-1)


def _node_tables(s, v, W_s1, b_s1, W_s2, b_s2, bN=400):
    N, H = s.shape
    grid = (N // bN,)
    return pl.pallas_call(
        _tables_body,
        grid=grid,
        in_specs=[
            pl.BlockSpec((bN, H), lambda i: (i, 0)),
            pl.BlockSpec((bN, 3, H), lambda i: (i, 0, 0)),
            pl.BlockSpec(W_s1.shape, lambda i: (0, 0)),
            pl.BlockSpec(b_s1.shape, lambda i: (0,)),
            pl.BlockSpec(W_s2.shape, lambda i: (0, 0)),
            pl.BlockSpec(b_s2.shape, lambda i: (0,)),
        ],
        out_specs=[
            pl.BlockSpec((2, bN, 256), lambda i: (0, i, 0)),
            pl.BlockSpec((2, bN, 256), lambda i: (0, i, 0)),
        ],
        out_shape=[
            jax.ShapeDtypeStruct((2, N, 256), jnp.float32),
            jax.ShapeDtypeStruct((2, N, 256), jnp.float32),
        ],
    )(s, v, W_s1, b_s1, W_s2, b_s2)


_K = 32


def _sc_body(N, Np, E, fa_hbm, fb_hbm, t1_hbm, t2_hbm, src_hbm, dst_hbm,
             u0_hbm, u12_hbm, zeros_hbm, out1_hbm, out2_hbm,
             acc,
             sbuf0, sbuf1, dbuf0, dbuf1, gidx0, gidx1, sidx0, sidx1,
             ubuf0, ubuf1, fabuf0, fabuf1, fbbuf0, fbbuf1,
             gbuf0, gbuf1, obuf0, obuf1,
             semi0, semi1, semg0, semg1, sems0, sems1):
    c = lax.axis_index("c")
    sid = lax.axis_index("s")
    K = _K
    ept = E // 16
    nwin = ept // K
    tile_lo = sid * ept
    coff_e = c * E

    rows = Np // 16
    row_lo = sid * rows
    cNp = c * Np

    cN_vec = jnp.full((16,), c * N, jnp.int32)

    sbuf = [sbuf0, sbuf1]
    dbuf = [dbuf0, dbuf1]
    gidx = [gidx0, gidx1]
    sidx = [sidx0, sidx1]
    ubuf = [ubuf0, ubuf1]
    fabuf = [fabuf0, fabuf1]
    fbbuf = [fbbuf0, fbbuf1]
    gbuf = [gbuf0, gbuf1]
    obuf = [obuf0, obuf1]
    semi = [semi0, semi1]
    semg = [semg0, semg1]
    sems = [sems0, sems1]

    def zero_acc():
        pltpu.sync_copy(zeros_hbm.at[pl.ds(row_lo, rows)],
                        acc.at[pl.ds(row_lo, rows)])
        plsc.subcore_barrier()

    def dump_acc(out_hbm):
        plsc.subcore_barrier()
        pltpu.sync_copy(acc.at[pl.ds(row_lo, rows)],
                        out_hbm.at[pl.ds(cNp + row_lo, rows)])
        plsc.subcore_barrier()

    def run_pass(tbl, out_hbm, first_pass, ebody):
        zero_acc()

        def in_copies(w, b):
            base = tile_lo + w * K
            cps = [
                (src_hbm.at[pl.ds(base, K)], sbuf[b]),
                (dst_hbm.at[pl.ds(base, K)], dbuf[b]),
                (fb_hbm.at[pl.ds(coff_e + base, K)], fbbuf[b]),
            ]
            if first_pass:
                cps.append((fa_hbm.at[pl.ds(coff_e + base, K)], fabuf[b]))
                cps.append((u0_hbm.at[pl.ds(base, K)], ubuf[b].at[pl.ds(0, K)]))
            else:
                cps.append((u12_hbm.at[pl.ds(2 * base, 2 * K)], ubuf[b]))
            return cps

        def fire_in(w, b):
            for s_, d_ in in_copies(w, b):
                pltpu.async_copy(s_, d_, semi[b])

        def drain_in(w, b):
            for s_, d_ in in_copies(w, b):
                pltpu.make_async_copy(s_, d_, semi[b]).wait()

        def prep_gather(b):
            for i in range(K // 16):
                sl = pl.ds(i * 16, 16)
                gidx[b][sl] = sbuf[b][sl] + cN_vec
            pltpu.async_copy(tbl.at[gidx[b]], gbuf[b], semg[b])

        def drain_gather(b):
            pltpu.make_async_copy(tbl.at[gidx[b]], gbuf[b], semg[b]).wait()

        def fire_scatter(b):
            for i in range(K // 16):
                sl = pl.ds(i * 16, 16)
                sidx[b][sl] = dbuf[b][sl]
            pltpu.async_copy(obuf[b], acc.at[sidx[b]], sems[b], add=True)

        def drain_scatter(b):
            pltpu.make_async_copy(obuf[b], acc.at[sidx[b]], sems[b]).wait()

        fire_in(0, 0)
        fire_in(1, 1)
        drain_in(0, 0)
        prep_gather(0)

        def wpbody(wp, carry):
            for half in range(2):
                w = wp * 2 + half
                b = half
                b1 = 1 - half

                @pl.when(w + 1 < nwin)
                def _():
                    drain_in(w + 1, b1)
                    prep_gather(b1)

                drain_gather(b)

                @pl.when(w >= 2)
                def _():
                    drain_scatter(b)

                lax.fori_loop(0, K, ebody(b), 0)
                fire_scatter(b)

                @pl.when(w + 2 < nwin)
                def _():
                    fire_in(w + 2, b)
            return carry

        lax.fori_loop(0, nwin // 2, wpbody, 0)
        drain_scatter(0)
        drain_scatter(1)
        dump_acc(out_hbm)

    def ebody1(b):
        def body(k, carry):
            u0 = plsc.load_gather(ubuf[b], [jnp.full((16,), 0, jnp.int32) + k])
            for j in range(4):
                sl = pl.ds(j * 16, 16)
                sh = pl.ds(64 + j * 16, 16)
                tds = gbuf[b][k, sl]
                g0 = gbuf[b][k, sh]
                tvr = gbuf[b][k, pl.ds(128 + j * 16, 16)]
                fds = fabuf[b][k, sl]
                fvv = fbbuf[b][k, sl]
                fvr = fbbuf[b][k, sh]
                mvr = fvr * tvr
                obuf[b][k, sl] = fds * tds
                obuf[b][k, sh] = fvv * g0 + mvr * u0
            return carry
        return body

    run_pass(t1_hbm, out1_hbm, True, ebody1)

    def ebody2(b):
        def body(k, carry):
            k2 = 2 * k
            u1 = plsc.load_gather(ubuf[b], [jnp.full((16,), 0, jnp.int32) + k2])
            u2 = plsc.load_gather(ubuf[b], [jnp.full((16,), 1, jnp.int32) + k2])
            for j in range(4):
                sl = pl.ds(j * 16, 16)
                sh = pl.ds(64 + j * 16, 16)
                g1 = gbuf[b][k, sl]
                g2 = gbuf[b][k, sh]
                tvr = gbuf[b][k, pl.ds(128 + j * 16, 16)]
                fvv = fbbuf[b][k, sl]
                fvr = fbbuf[b][k, sh]
                mvr = fvr * tvr
                obuf[b][k, sl] = fvv * g1 + mvr * u1
                obuf[b][k, sh] = fvv * g2 + mvr * u2
            return carry
        return body

    run_pass(t2_hbm, out2_hbm, False, ebody2)


def _sc_scatter(fa, fb, t1, t2, src, dst, u0, u12, zeros, N, Np, E):
    mesh = plsc.VectorSubcoreMesh(core_axis_name="c", subcore_axis_name="s")
    K = _K
    dbl = lambda mk: [mk(), mk()]
    kfn = functools.partial(
        pl.kernel,
        out_type=[
            jax.ShapeDtypeStruct((2 * Np, 128), jnp.float32),
            jax.ShapeDtypeStruct((2 * Np, 128), jnp.float32),
        ],
        mesh=mesh,
        scratch_types=(
            [pltpu.VMEM_SHARED((Np, 128), jnp.float32)]
            + dbl(lambda: pltpu.VMEM((K,), jnp.int32))
            + dbl(lambda: pltpu.VMEM((K,), jnp.int32))
            + dbl(lambda: pltpu.VMEM((K,), jnp.int32))
            + dbl(lambda: pltpu.VMEM((K,), jnp.int32))
            + dbl(lambda: pltpu.VMEM((2 * K,), jnp.float32))
            + dbl(lambda: pltpu.VMEM((K, 64), jnp.float32))
            + dbl(lambda: pltpu.VMEM((K, 128), jnp.float32))
            + dbl(lambda: pltpu.VMEM((K, 256), jnp.float32))
            + dbl(lambda: pltpu.VMEM((K, 128), jnp.float32))
            + [pltpu.SemaphoreType.DMA] * 6
        ),
        compiler_params=pltpu.CompilerParams(needs_layout_passes=False),
    )(functools.partial(_sc_body, N, Np, E))
    return kfn(fa, fb, t1, t2, src, dst, u0, u12, zeros)


def _update_body(s_ref, v_ref, o1_ref, o2_ref, uw_ref, vw_ref,
                 wu1_ref, bu1_ref, wu2_ref, bu2_ref, s_out_ref, v_out_ref):
    bN, _, H = v_ref.shape
    ds = jnp.concatenate([o1_ref[0][:, 0:64], o1_ref[1][:, 0:64]], axis=-1)
    dv0 = jnp.concatenate([o1_ref[0][:, 64:128], o1_ref[1][:, 64:128]], axis=-1)
    dv1 = jnp.concatenate([o2_ref[0][:, 0:64], o2_ref[1][:, 0:64]], axis=-1)
    dv2 = jnp.concatenate([o2_ref[0][:, 64:128], o2_ref[1][:, 64:128]], axis=-1)
    dv = jnp.concatenate([dv0[:, None, :], dv1[:, None, :], dv2[:, None, :]],
                         axis=1)
    s1 = s_ref[...] + ds
    v1 = v_ref[...] + dv
    v1f = v1.reshape(bN * 3, H)
    v_u = jnp.dot(v1f, uw_ref[...], preferred_element_type=jnp.float32)
    v_v = jnp.dot(v1f, vw_ref[...], preferred_element_type=jnp.float32)
    v_u = v_u.reshape(bN, 3, H)
    v_v = v_v.reshape(bN, 3, H)
    v_norm = jnp.sqrt(jnp.sum(v_v * v_v, axis=1))
    upd_in = jnp.concatenate([s1, v_norm], axis=-1)
    h = _silu(jnp.dot(upd_in, wu1_ref[...],
                      preferred_element_type=jnp.float32) + bu1_ref[...])
    out = jnp.dot(h, wu2_ref[...], preferred_element_type=jnp.float32) + bu2_ref[...]
    a = out[:, :H]
    b = out[:, H:2 * H]
    cc = out[:, 2 * H:]
    inner = jnp.sum(v_u * v_v, axis=1)
    s_out_ref[...] = s1 + a + b * inner
    v_out_ref[...] = v1 + cc[:, None, :] * v_u


def _update_phase(s, v, o1, o2, U_w, V_w, W_u1, b_u1, W_u2, b_u2, bN=400):
    N, H = s.shape
    grid = (N // bN,)
    return pl.pallas_call(
        _update_body,
        grid=grid,
        in_specs=[
            pl.BlockSpec((bN, H), lambda i: (i, 0)),
            pl.BlockSpec((bN, 3, H), lambda i: (i, 0, 0)),
            pl.BlockSpec((2, bN, 128), lambda i: (0, i, 0)),
            pl.BlockSpec((2, bN, 128), lambda i: (0, i, 0)),
            pl.BlockSpec(U_w.shape, lambda i: (0, 0)),
            pl.BlockSpec(V_w.shape, lambda i: (0, 0)),
            pl.BlockSpec(W_u1.shape, lambda i: (0, 0)),
            pl.BlockSpec(b_u1.shape, lambda i: (0,)),
            pl.BlockSpec(W_u2.shape, lambda i: (0, 0)),
            pl.BlockSpec(b_u2.shape, lambda i: (0,)),
        ],
        out_specs=[
            pl.BlockSpec((bN, H), lambda i: (i, 0)),
            pl.BlockSpec((bN, 3, H), lambda i: (i, 0, 0)),
        ],
        out_shape=[
            jax.ShapeDtypeStruct((N, H), jnp.float32),
            jax.ShapeDtypeStruct((N, 3, H), jnp.float32),
        ],
    )(s, v, o1, o2, U_w, V_w, W_u1, b_u1, W_u2, b_u2)


def kernel(s, v, edge_index, rbf, unit,
           W_f1, b_f1, W_f2, b_f2,
           W_s1, b_s1, W_s2, b_s2,
           U_w, V_w, W_u1, b_u1, W_u2, b_u2):
    N, H = s.shape
    E = edge_index.shape[1]
    src = edge_index[0]
    dst = edge_index[1]

    perm = jnp.concatenate([
        jnp.arange(0, 128), jnp.arange(128, 192), jnp.arange(256, 320),
        jnp.arange(192, 256), jnp.arange(320, 384)])
    W_f2p = W_f2[:, perm]
    b_f2p = b_f2[perm]

    Np = 10240
    Ep = 327680
    npad = Ep - E
    rbf_p = jnp.pad(rbf, ((0, npad), (0, 0)))
    src_p = jnp.concatenate([src, jnp.arange(npad, dtype=jnp.int32) % N])
    dst_p = jnp.concatenate(
        [dst, N + (jnp.arange(npad, dtype=jnp.int32) % (Np - N))])
    unit_p = jnp.pad(unit, ((0, npad), (0, 0)))
    u0 = unit_p[:, 0]
    u12 = unit_p[:, 1:3].reshape(2 * Ep)

    fa, fb = _filter_mlp(rbf_p, W_f1, b_f1, W_f2p, b_f2p)
    t1, t2 = _node_tables(s, v, W_s1, b_s1, W_s2, b_s2)

    fa = fa.reshape(2 * Ep, 64)
    fb = fb.reshape(2 * Ep, 128)
    t1 = t1.reshape(2 * N, 256)
    t2 = t2.reshape(2 * N, 256)
    zeros = jnp.zeros((Np, 128), jnp.float32)

    o1, o2 = _sc_scatter(fa, fb, t1, t2, src_p, dst_p, u0, u12, zeros, N, Np, Ep)
    o1 = o1.reshape(2, Np, 128)
    o2 = o2.reshape(2, Np, 128)

    return _update_phase(s, v, o1, o2, U_w, V_w, W_u1, b_u1, W_u2, b_u2)
